# baseline (speedup 1.0000x reference)
import functools

import jax
import jax.numpy as jnp
from jax import lax
from jax.experimental import pallas as pl
from jax.experimental.pallas import tpu as pltpu
from jax.experimental.pallas import tpu_sc as plsc

N = 10000
NPAD = 10240
E = 160000
EPAD = 163840
IN_F = 128
MLP_H = 512
PROJ = 256
FC = 128
NC = 2
NS = 16
CH = 128
ROWS_W = NPAD // NS
W32_CHUNKS = EPAD // (NS * NC * CH)
EPW32 = EPAD // (NS * NC)

RB = 1000
GRID_R = N // RB

_MESH = plsc.VectorSubcoreMesh(core_axis_name="c", subcore_axis_name="s",
                               num_cores=NC, num_subcores=NS)


@functools.partial(
    pl.kernel,
    out_type=jax.ShapeDtypeStruct((NC * NS, NPAD), jnp.float32),
    mesh=_MESH,
    scratch_types=[
        pltpu.VMEM((EPW32,), jnp.int32),
        pltpu.VMEM((NPAD,), jnp.float32),
    ],
    name="sc_degree",
    compiler_params=pltpu.CompilerParams(needs_layout_passes=False),
)
def _deg_kernel(dst_hbm, out_hbm, dstl, hist):
    c = lax.axis_index("c")
    s = lax.axis_index("s")
    w = s * NC + c
    pltpu.sync_copy(dst_hbm.at[pl.ds(w * EPW32, EPW32)], dstl)

    def zbody(i, carry):
        hist[pl.ds(i * 16, 16)] = jnp.zeros((16,), jnp.float32)
        return carry

    lax.fori_loop(0, NPAD // 16, zbody, 0)
    ones16 = jnp.ones((16,), jnp.float32)

    def body(i, carry):
        idx = dstl[pl.ds(i * 16, 16)]
        plsc.addupdate_scatter(hist, [idx], ones16)
        return carry

    lax.fori_loop(0, EPW32 // 16, body, 0)
    pltpu.sync_copy(hist, out_hbm.at[w])


KBUF = 2


def _edge_pipeline(u_view, acc, srcl, dstl, rows, gsems, ssem, n_chunks, ch):
    def body(j, carry):
        gds = []
        for k in range(KBUF):
            i = j * KBUF + k
            gds.append(pltpu.async_copy(u_view.at[srcl.at[i]],
                                        rows.at[pl.ds(k * ch, ch)],
                                        gsems[k]))
        sds = []
        for k in range(KBUF):
            i = j * KBUF + k
            gds[k].wait()
            sds.append(pltpu.async_copy(rows.at[pl.ds(k * ch, ch)],
                                        acc.at[dstl.at[i]], ssem,
                                        add=True))
        for k in range(KBUF):
            sds[k].wait()
        return carry

    lax.fori_loop(0, n_chunks // KBUF, body, 0)


@functools.partial(
    pl.kernel,
    out_type=jax.ShapeDtypeStruct((NC, NPAD, FC), jnp.float32),
    mesh=_MESH,
    scratch_types=[
        pltpu.VMEM((W32_CHUNKS, CH), jnp.int32),
        pltpu.VMEM((W32_CHUNKS, CH), jnp.int32),
        pltpu.VMEM((KBUF * CH, FC), jnp.float32),
        pltpu.VMEM_SHARED((NPAD, FC), jnp.float32),
        [pltpu.SemaphoreType.DMA] * KBUF,
        pltpu.SemaphoreType.DMA,
    ],
    name="sc_prop_enc",
    compiler_params=pltpu.CompilerParams(needs_layout_passes=False),
)
def _prop_enc(u_hbm, src_hbm, dst_hbm, out_hbm, srcl, dstl, rows, acc,
              gsems, ssem):
    c = lax.axis_index("c")
    s = lax.axis_index("s")
    w = s * NC + c
    pltpu.sync_copy(src_hbm.at[pl.ds(w * W32_CHUNKS, W32_CHUNKS)], srcl)
    pltpu.sync_copy(dst_hbm.at[pl.ds(w * W32_CHUNKS, W32_CHUNKS)], dstl)
    pltpu.sync_copy(u_hbm.at[pl.ds(s * ROWS_W, ROWS_W)],
                    acc.at[pl.ds(s * ROWS_W, ROWS_W)])
    plsc.subcore_barrier()
    _edge_pipeline(u_hbm, acc, srcl, dstl, rows, gsems, ssem, W32_CHUNKS, CH)
    plsc.subcore_barrier()
    pltpu.sync_copy(acc.at[pl.ds(s * ROWS_W, ROWS_W)],
                    out_hbm.at[c].at[pl.ds(s * ROWS_W, ROWS_W)])


def _tca_body(degp_ref, x_ref, dinv_ref, dinv2_ref, u1_ref):
    deg = jnp.sum(degp_ref[...], axis=0) + 1.0
    dinv = lax.rsqrt(jnp.maximum(deg, 1e-12))
    dinv_ref[...] = dinv
    dinv2_ref[...] = dinv * dinv
    u1_ref[...] = x_ref[...] * dinv


_tc_a = pl.pallas_call(
    _tca_body,
    grid=(GRID_R,),
    in_specs=[
        pl.BlockSpec((NC * NS, RB, 1), lambda i: (0, i, 0)),
        pl.BlockSpec((RB, IN_F), lambda i: (i, 0)),
    ],
    out_specs=[
        pl.BlockSpec((RB, 1), lambda i: (i, 0)),
        pl.BlockSpec((RB, 1), lambda i: (i, 0)),
        pl.BlockSpec((RB, IN_F), lambda i: (i, 0)),
    ],
    out_shape=[
        jax.ShapeDtypeStruct((N, 1), jnp.float32),
        jax.ShapeDtypeStruct((N, 1), jnp.float32),
        jax.ShapeDtypeStruct((NPAD, IN_F), jnp.float32),
    ],
)


def _scale_enc_body(s_ref, u_ref, d2_ref, o_ref):
    o_ref[...] = (s_ref[0] + s_ref[1] - u_ref[...]) * d2_ref[...]


_scale_enc = pl.pallas_call(
    _scale_enc_body,
    grid=(GRID_R,),
    in_specs=[
        pl.BlockSpec((NC, RB, FC), lambda i: (0, i, 0)),
        pl.BlockSpec((RB, FC), lambda i: (i, 0)),
        pl.BlockSpec((RB, 1), lambda i: (i, 0)),
    ],
    out_specs=pl.BlockSpec((RB, FC), lambda i: (i, 0)),
    out_shape=jax.ShapeDtypeStruct((NPAD, FC), jnp.float32),
)


def _tcc_body(s2_ref, u2_ref, dinv_ref, encW_ref, encb_ref, W1_ref, b1_ref,
              hh_ref, stats_ref):
    i = pl.program_id(0)
    p = (s2_ref[0] + s2_ref[1] - u2_ref[...]) * dinv_ref[...]
    h = jnp.dot(p, encW_ref[...], preferred_element_type=jnp.float32) \
        + encb_ref[...]
    hh = jnp.dot(h, W1_ref[...], preferred_element_type=jnp.float32) \
        + b1_ref[...]
    hh_ref[...] = hh

    @pl.when(i == 0)
    def _init():
        stats_ref[...] = jnp.zeros((2, MLP_H), jnp.float32)

    stats_ref[0:1] += jnp.sum(hh, axis=0, keepdims=True)
    stats_ref[1:2] += jnp.sum(hh * hh, axis=0, keepdims=True)


_tc_c = pl.pallas_call(
    _tcc_body,
    grid=(GRID_R,),
    in_specs=[
        pl.BlockSpec((NC, RB, FC), lambda i: (0, i, 0)),
        pl.BlockSpec((RB, FC), lambda i: (i, 0)),
        pl.BlockSpec((RB, 1), lambda i: (i, 0)),
        pl.BlockSpec((IN_F, IN_F), lambda i: (0, 0)),
        pl.BlockSpec((1, IN_F), lambda i: (0, 0)),
        pl.BlockSpec((IN_F, MLP_H), lambda i: (0, 0)),
        pl.BlockSpec((1, MLP_H), lambda i: (0, 0)),
    ],
    out_specs=[
        pl.BlockSpec((RB, MLP_H), lambda i: (i, 0)),
        pl.BlockSpec((2, MLP_H), lambda i: (0, 0)),
    ],
    out_shape=[
        jax.ShapeDtypeStruct((N, MLP_H), jnp.float32),
        jax.ShapeDtypeStruct((2, MLP_H), jnp.float32),
    ],
)


def _tcd_body(hh_ref, stats_ref, gamma_ref, beta_ref, W2_ref, b2_ref,
              dinv_ref, z_ref, u3a_ref, u3b_ref):
    hh = hh_ref[...]
    inv_n = jnp.float32(1.0 / N)
    mean = stats_ref[0:1] * inv_n
    var = stats_ref[1:2] * inv_n - mean * mean
    scale = lax.rsqrt(var + 1e-5) * gamma_ref[...]
    shift = beta_ref[...] - mean * scale
    a = jnp.maximum(hh * scale + shift, 0.0)
    z = jnp.dot(a, W2_ref[...], preferred_element_type=jnp.float32) \
        + b2_ref[...]
    z_ref[...] = z
    u = z * dinv_ref[...]
    u3a_ref[...] = u[:, :FC]
    u3b_ref[...] = u[:, FC:]


_tc_d = pl.pallas_call(
    _tcd_body,
    grid=(GRID_R,),
    in_specs=[
        pl.BlockSpec((RB, MLP_H), lambda i: (i, 0)),
        pl.BlockSpec((2, MLP_H), lambda i: (0, 0)),
        pl.BlockSpec((1, MLP_H), lambda i: (0, 0)),
        pl.BlockSpec((1, MLP_H), lambda i: (0, 0)),
        pl.BlockSpec((MLP_H, PROJ), lambda i: (0, 0)),
        pl.BlockSpec((1, PROJ), lambda i: (0, 0)),
        pl.BlockSpec((RB, 1), lambda i: (i, 0)),
    ],
    out_specs=[
        pl.BlockSpec((RB, PROJ), lambda i: (i, 0)),
        pl.BlockSpec((RB, FC), lambda i: (i, 0)),
        pl.BlockSpec((RB, FC), lambda i: (i, 0)),
    ],
    out_shape=[
        jax.ShapeDtypeStruct((N, PROJ), jnp.float32),
        jax.ShapeDtypeStruct((NPAD, FC), jnp.float32),
        jax.ShapeDtypeStruct((NPAD, FC), jnp.float32),
    ],
)


def _tcf_body(s4a_ref, u4a_ref, s4b_ref, u4b_ref, dinv_ref, decW_ref,
              decb_ref, o_ref):
    qa = s4a_ref[0] + s4a_ref[1] - u4a_ref[...]
    qb = s4b_ref[0] + s4b_ref[1] - u4b_ref[...]
    q = jnp.concatenate([qa, qb], axis=1) * dinv_ref[...]
    o_ref[...] = jnp.dot(q, decW_ref[...], preferred_element_type=jnp.float32) \
        + decb_ref[...]


_tc_f = pl.pallas_call(
    _tcf_body,
    grid=(GRID_R,),
    in_specs=[
        pl.BlockSpec((NC, RB, FC), lambda i: (0, i, 0)),
        pl.BlockSpec((RB, FC), lambda i: (i, 0)),
        pl.BlockSpec((NC, RB, FC), lambda i: (0, i, 0)),
        pl.BlockSpec((RB, FC), lambda i: (i, 0)),
        pl.BlockSpec((RB, 1), lambda i: (i, 0)),
        pl.BlockSpec((PROJ, IN_F), lambda i: (0, 0)),
        pl.BlockSpec((1, IN_F), lambda i: (0, 0)),
    ],
    out_specs=pl.BlockSpec((RB, IN_F), lambda i: (i, 0)),
    out_shape=jax.ShapeDtypeStruct((N, IN_F), jnp.float32),
)


def _ahat_body(za_ref, zb_ref, o_ref):
    o_ref[...] = lax.dot_general(
        za_ref[...], zb_ref[...],
        (((1,), (1,)), ((), ())),
        preferred_element_type=jnp.float32)


MB = 1024
GRID_M = (N + MB - 1) // MB

_ahat = pl.pallas_call(
    _ahat_body,
    grid=(GRID_M, GRID_M),
    in_specs=[
        pl.BlockSpec((MB, PROJ), lambda i, j: (i, 0)),
        pl.BlockSpec((MB, PROJ), lambda i, j: (j, 0)),
    ],
    out_specs=pl.BlockSpec((MB, MB), lambda i, j: (i, j)),
    out_shape=jax.ShapeDtypeStruct((N, N), jnp.float32),
)


def kernel(x, edge_index, enc_W, enc_b, W1, b1, gamma, beta, W2, b2,
           dec_W, dec_b):
    ei = edge_index.astype(jnp.int32)
    src = jnp.concatenate([ei[0], jnp.zeros((EPAD - E,), jnp.int32)])
    dst = jnp.concatenate([ei[1], jnp.full((EPAD - E,), N, jnp.int32)])
    src2 = src.reshape(NS * NC * W32_CHUNKS, CH)
    dst2 = dst.reshape(NS * NC * W32_CHUNKS, CH)

    degp = _deg_kernel(dst)
    dinv, dinv2, u1 = _tc_a(degp.reshape(NC * NS, NPAD, 1), x)
    s1 = _prop_enc(u1, src2, dst2)
    u2 = _scale_enc(s1, u1, dinv2)
    s2 = _prop_enc(u2, src2, dst2)
    hh, stats = _tc_c(s2, u2, dinv, enc_W, enc_b.reshape(1, -1),
                      W1, b1.reshape(1, -1))
    z, u3a, u3b = _tc_d(hh, stats, gamma.reshape(1, -1),
                        beta.reshape(1, -1), W2, b2.reshape(1, -1), dinv)
    s3a = _prop_enc(u3a, src2, dst2)
    s3b = _prop_enc(u3b, src2, dst2)
    u4a = _scale_enc(s3a, u3a, dinv2)
    u4b = _scale_enc(s3b, u3b, dinv2)
    s4a = _prop_enc(u4a, src2, dst2)
    s4b = _prop_enc(u4b, src2, dst2)
    x_hat = _tc_f(s4a, u4a, s4b, u4b, dinv, dec_W, dec_b.reshape(1, -1))
    a_hat = _ahat(z, z)
    return (x_hat, a_hat)


# speedup vs baseline: 1.3697x; 1.3697x over previous
import functools

import jax
import jax.numpy as jnp
from jax import lax
from jax.experimental import pallas as pl
from jax.experimental.pallas import tpu as pltpu
from jax.experimental.pallas import tpu_sc as plsc

N = 10000
NPAD = 10240
E = 160000
EPAD = 163840
IN_F = 128
MLP_H = 512
PROJ = 256
FC = 128
NC = 2
NS = 16
CH = 128
ROWS_W = NPAD // NS
W32_CHUNKS = EPAD // (NS * NC * CH)
EPW32 = EPAD // (NS * NC)

RB = 1000
GRID_R = N // RB

_MESH = plsc.VectorSubcoreMesh(core_axis_name="c", subcore_axis_name="s",
                               num_cores=NC, num_subcores=NS)


@functools.partial(
    pl.kernel,
    out_type=jax.ShapeDtypeStruct((NC * NS, NPAD), jnp.float32),
    mesh=_MESH,
    scratch_types=[
        pltpu.VMEM((EPW32,), jnp.int32),
        pltpu.VMEM((NPAD,), jnp.float32),
    ],
    name="sc_degree",
    compiler_params=pltpu.CompilerParams(needs_layout_passes=False),
)
def _deg_kernel(dst_hbm, out_hbm, dstl, hist):
    c = lax.axis_index("c")
    s = lax.axis_index("s")
    w = s * NC + c
    pltpu.sync_copy(dst_hbm.at[pl.ds(w * EPW32, EPW32)], dstl)

    def zbody(i, carry):
        hist[pl.ds(i * 16, 16)] = jnp.zeros((16,), jnp.float32)
        return carry

    lax.fori_loop(0, NPAD // 16, zbody, 0)
    ones16 = jnp.ones((16,), jnp.float32)

    def body(i, carry):
        idx = dstl[pl.ds(i * 16, 16)]
        plsc.addupdate_scatter(hist, [idx], ones16)
        return carry

    lax.fori_loop(0, EPW32 // 16, body, 0)
    pltpu.sync_copy(hist, out_hbm.at[w])


KBUF = 2


def _edge_pipeline(u_view, acc, srcl, dstl, rows, gsems, ssem, n_chunks, ch):
    def body(j, carry):
        gds = []
        for k in range(KBUF):
            i = j * KBUF + k
            gds.append(pltpu.async_copy(u_view.at[srcl.at[i]],
                                        rows.at[pl.ds(k * ch, ch)],
                                        gsems[k]))
        sds = []
        for k in range(KBUF):
            i = j * KBUF + k
            gds[k].wait()
            sds.append(pltpu.async_copy(rows.at[pl.ds(k * ch, ch)],
                                        acc.at[dstl.at[i]], ssem,
                                        add=True))
        for k in range(KBUF):
            sds[k].wait()
        return carry

    lax.fori_loop(0, n_chunks // KBUF, body, 0)


@functools.partial(
    pl.kernel,
    out_type=jax.ShapeDtypeStruct((NC, NPAD, FC), jnp.float32),
    mesh=_MESH,
    scratch_types=[
        pltpu.VMEM((W32_CHUNKS, CH), jnp.int32),
        pltpu.VMEM((W32_CHUNKS, CH), jnp.int32),
        pltpu.VMEM((KBUF * CH, FC), jnp.float32),
        pltpu.VMEM_SHARED((NPAD, FC), jnp.float32),
        [pltpu.SemaphoreType.DMA] * KBUF,
        pltpu.SemaphoreType.DMA,
    ],
    name="sc_prop_enc",
    compiler_params=pltpu.CompilerParams(needs_layout_passes=False),
)
def _prop_enc(u_hbm, src_hbm, dst_hbm, out_hbm, srcl, dstl, rows, acc,
              gsems, ssem):
    c = lax.axis_index("c")
    s = lax.axis_index("s")
    w = s * NC + c
    pltpu.sync_copy(src_hbm.at[pl.ds(w * W32_CHUNKS, W32_CHUNKS)], srcl)
    pltpu.sync_copy(dst_hbm.at[pl.ds(w * W32_CHUNKS, W32_CHUNKS)], dstl)
    pltpu.sync_copy(u_hbm.at[pl.ds(s * ROWS_W, ROWS_W)],
                    acc.at[pl.ds(s * ROWS_W, ROWS_W)])
    plsc.subcore_barrier()
    _edge_pipeline(u_hbm, acc, srcl, dstl, rows, gsems, ssem, W32_CHUNKS, CH)
    plsc.subcore_barrier()
    pltpu.sync_copy(acc.at[pl.ds(s * ROWS_W, ROWS_W)],
                    out_hbm.at[c].at[pl.ds(s * ROWS_W, ROWS_W)])


def _tca_body(degp_ref, x_ref, dinv_ref, dinv2_ref, u1_ref):
    deg = jnp.sum(degp_ref[...], axis=0) + 1.0
    dinv = lax.rsqrt(jnp.maximum(deg, 1e-12))
    dinv_ref[...] = dinv
    dinv2_ref[...] = dinv * dinv
    u1_ref[...] = x_ref[...] * dinv


_tc_a = pl.pallas_call(
    _tca_body,
    grid=(GRID_R,),
    in_specs=[
        pl.BlockSpec((NC * NS, RB, 1), lambda i: (0, i, 0)),
        pl.BlockSpec((RB, IN_F), lambda i: (i, 0)),
    ],
    out_specs=[
        pl.BlockSpec((RB, 1), lambda i: (i, 0)),
        pl.BlockSpec((RB, 1), lambda i: (i, 0)),
        pl.BlockSpec((RB, IN_F), lambda i: (i, 0)),
    ],
    out_shape=[
        jax.ShapeDtypeStruct((N, 1), jnp.float32),
        jax.ShapeDtypeStruct((N, 1), jnp.float32),
        jax.ShapeDtypeStruct((NPAD, IN_F), jnp.float32),
    ],
)


def _scale_enc_body(s_ref, u_ref, d2_ref, o_ref):
    o_ref[...] = (s_ref[0] + s_ref[1] - u_ref[...]) * d2_ref[...]


_scale_enc = pl.pallas_call(
    _scale_enc_body,
    grid=(GRID_R,),
    in_specs=[
        pl.BlockSpec((NC, RB, FC), lambda i: (0, i, 0)),
        pl.BlockSpec((RB, FC), lambda i: (i, 0)),
        pl.BlockSpec((RB, 1), lambda i: (i, 0)),
    ],
    out_specs=pl.BlockSpec((RB, FC), lambda i: (i, 0)),
    out_shape=jax.ShapeDtypeStruct((NPAD, FC), jnp.float32),
)


def _tcc_body(s2_ref, u2_ref, dinv_ref, encW_ref, encb_ref, W1_ref, b1_ref,
              hh_ref, stats_ref):
    i = pl.program_id(0)
    p = (s2_ref[0] + s2_ref[1] - u2_ref[...]) * dinv_ref[...]
    h = jnp.dot(p, encW_ref[...], preferred_element_type=jnp.float32) \
        + encb_ref[...]
    hh = jnp.dot(h, W1_ref[...], preferred_element_type=jnp.float32) \
        + b1_ref[...]
    hh_ref[...] = hh

    @pl.when(i == 0)
    def _init():
        stats_ref[...] = jnp.zeros((2, MLP_H), jnp.float32)

    stats_ref[0:1] += jnp.sum(hh, axis=0, keepdims=True)
    stats_ref[1:2] += jnp.sum(hh * hh, axis=0, keepdims=True)


_tc_c = pl.pallas_call(
    _tcc_body,
    grid=(GRID_R,),
    in_specs=[
        pl.BlockSpec((NC, RB, FC), lambda i: (0, i, 0)),
        pl.BlockSpec((RB, FC), lambda i: (i, 0)),
        pl.BlockSpec((RB, 1), lambda i: (i, 0)),
        pl.BlockSpec((IN_F, IN_F), lambda i: (0, 0)),
        pl.BlockSpec((1, IN_F), lambda i: (0, 0)),
        pl.BlockSpec((IN_F, MLP_H), lambda i: (0, 0)),
        pl.BlockSpec((1, MLP_H), lambda i: (0, 0)),
    ],
    out_specs=[
        pl.BlockSpec((RB, MLP_H), lambda i: (i, 0)),
        pl.BlockSpec((2, MLP_H), lambda i: (0, 0)),
    ],
    out_shape=[
        jax.ShapeDtypeStruct((N, MLP_H), jnp.float32),
        jax.ShapeDtypeStruct((2, MLP_H), jnp.float32),
    ],
)


def _tcd_body(hh_ref, stats_ref, gamma_ref, beta_ref, W2_ref, b2_ref,
              decW_ref, dinv_ref, z_ref, u3_ref):
    hh = hh_ref[...]
    inv_n = jnp.float32(1.0 / N)
    mean = stats_ref[0:1] * inv_n
    var = stats_ref[1:2] * inv_n - mean * mean
    scale = lax.rsqrt(var + 1e-5) * gamma_ref[...]
    shift = beta_ref[...] - mean * scale
    a = jnp.maximum(hh * scale + shift, 0.0)
    z = jnp.dot(a, W2_ref[...], preferred_element_type=jnp.float32) \
        + b2_ref[...]
    z_ref[...] = z
    y = jnp.dot(z, decW_ref[...], preferred_element_type=jnp.float32)
    u3_ref[...] = y * dinv_ref[...]


_tc_d = pl.pallas_call(
    _tcd_body,
    grid=(GRID_R,),
    in_specs=[
        pl.BlockSpec((RB, MLP_H), lambda i: (i, 0)),
        pl.BlockSpec((2, MLP_H), lambda i: (0, 0)),
        pl.BlockSpec((1, MLP_H), lambda i: (0, 0)),
        pl.BlockSpec((1, MLP_H), lambda i: (0, 0)),
        pl.BlockSpec((MLP_H, PROJ), lambda i: (0, 0)),
        pl.BlockSpec((1, PROJ), lambda i: (0, 0)),
        pl.BlockSpec((PROJ, IN_F), lambda i: (0, 0)),
        pl.BlockSpec((RB, 1), lambda i: (i, 0)),
    ],
    out_specs=[
        pl.BlockSpec((RB, PROJ), lambda i: (i, 0)),
        pl.BlockSpec((RB, FC), lambda i: (i, 0)),
    ],
    out_shape=[
        jax.ShapeDtypeStruct((N, PROJ), jnp.float32),
        jax.ShapeDtypeStruct((NPAD, FC), jnp.float32),
    ],
)


def _tcf_body(s4_ref, u4_ref, dinv_ref, decb_ref, o_ref):
    q = s4_ref[0] + s4_ref[1] - u4_ref[...]
    o_ref[...] = q * dinv_ref[...] + decb_ref[...]


_tc_f = pl.pallas_call(
    _tcf_body,
    grid=(GRID_R,),
    in_specs=[
        pl.BlockSpec((NC, RB, FC), lambda i: (0, i, 0)),
        pl.BlockSpec((RB, FC), lambda i: (i, 0)),
        pl.BlockSpec((RB, 1), lambda i: (i, 0)),
        pl.BlockSpec((1, IN_F), lambda i: (0, 0)),
    ],
    out_specs=pl.BlockSpec((RB, IN_F), lambda i: (i, 0)),
    out_shape=jax.ShapeDtypeStruct((N, IN_F), jnp.float32),
)


def _ahat_body(za_ref, zb_ref, o_ref):
    o_ref[...] = lax.dot_general(
        za_ref[...], zb_ref[...],
        (((1,), (1,)), ((), ())),
        preferred_element_type=jnp.float32)


MB = 1024
GRID_M = (N + MB - 1) // MB

_ahat = pl.pallas_call(
    _ahat_body,
    grid=(GRID_M, GRID_M),
    in_specs=[
        pl.BlockSpec((MB, PROJ), lambda i, j: (i, 0)),
        pl.BlockSpec((MB, PROJ), lambda i, j: (j, 0)),
    ],
    out_specs=pl.BlockSpec((MB, MB), lambda i, j: (i, j)),
    out_shape=jax.ShapeDtypeStruct((N, N), jnp.float32),
)


def kernel(x, edge_index, enc_W, enc_b, W1, b1, gamma, beta, W2, b2,
           dec_W, dec_b):
    ei = edge_index.astype(jnp.int32)
    src = jnp.concatenate([ei[0], jnp.zeros((EPAD - E,), jnp.int32)])
    dst = jnp.concatenate([ei[1], jnp.full((EPAD - E,), N, jnp.int32)])
    src2 = src.reshape(NS * NC * W32_CHUNKS, CH)
    dst2 = dst.reshape(NS * NC * W32_CHUNKS, CH)

    degp = _deg_kernel(dst)
    dinv, dinv2, u1 = _tc_a(degp.reshape(NC * NS, NPAD, 1), x)
    s1 = _prop_enc(u1, src2, dst2)
    u2 = _scale_enc(s1, u1, dinv2)
    s2 = _prop_enc(u2, src2, dst2)
    hh, stats = _tc_c(s2, u2, dinv, enc_W, enc_b.reshape(1, -1),
                      W1, b1.reshape(1, -1))
    z, u3 = _tc_d(hh, stats, gamma.reshape(1, -1), beta.reshape(1, -1),
                  W2, b2.reshape(1, -1), dec_W, dinv)
    s3 = _prop_enc(u3, src2, dst2)
    u4 = _scale_enc(s3, u3, dinv2)
    s4 = _prop_enc(u4, src2, dst2)
    x_hat = _tc_f(s4, u4, dinv, dec_b.reshape(1, -1))
    a_hat = _ahat(z, z)
    return (x_hat, a_hat)
